# baseline (speedup 1.0000x reference)
import jax
import jax.numpy as jnp
from jax.experimental import pallas as pl
from jax.experimental.pallas import tpu as pltpu

_S = 2
_BM = 256


def _matmul_block(*refs):
    a_refs = refs[:_S]
    f_ref = refs[_S]
    o_ref = refs[_S + 1]
    f = f_ref[...].astype(jnp.bfloat16)
    for s in range(_S):
        a = a_refs[s][0].astype(jnp.bfloat16)
        o_ref[s] = jnp.dot(a, f, preferred_element_type=jnp.float32)


@jax.jit
def kernel(features, A):
    if features.ndim != 2:
        raise RuntimeError('the dimension of features should be 2')
    M, K = A.shape
    _, N = features.shape
    Ar = A.reshape(_S, M // _S, K)

    def a_spec(s):
        return pl.BlockSpec((1, _BM, K), lambda i, s=s: (s, i, 0))

    out = pl.pallas_call(
        _matmul_block,
        grid=(M // _S // _BM,),
        in_specs=[a_spec(s) for s in range(_S)]
        + [pl.BlockSpec((K, N), lambda i: (0, 0))],
        out_specs=pl.BlockSpec((_S, _BM, N), lambda i: (0, i, 0)),
        out_shape=jax.ShapeDtypeStruct((_S, M // _S, N), jnp.float32),
        compiler_params=pltpu.CompilerParams(
            dimension_semantics=("arbitrary",),
        ),
    )(*([Ar] * _S), features)
    return out.reshape(M, N)


# speedup vs baseline: 1.0421x; 1.0421x over previous
import jax
import jax.numpy as jnp
from jax.experimental import pallas as pl
from jax.experimental.pallas import tpu as pltpu


def _matmul_block(a_ref, f_ref, o_ref):
    a = a_ref[...].astype(jnp.bfloat16)
    f = f_ref[...].astype(jnp.bfloat16)
    o_ref[...] = jnp.dot(a, f, preferred_element_type=jnp.float32)


@jax.jit
def kernel(features, A):
    if features.ndim != 2:
        raise RuntimeError('the dimension of features should be 2')
    M, K = A.shape
    _, N = features.shape
    BM = 256
    return pl.pallas_call(
        _matmul_block,
        grid=(M // BM,),
        in_specs=[
            pl.BlockSpec((BM, K), lambda i: (i, 0)),
            pl.BlockSpec((K, N), lambda i: (0, 0)),
        ],
        out_specs=pl.BlockSpec((BM, N), lambda i: (i, 0)),
        out_shape=jax.ShapeDtypeStruct((M, N), jnp.float32),
        compiler_params=pltpu.CompilerParams(
            dimension_semantics=("parallel",),
        ),
    )(A, features)
